# baseline (speedup 1.0000x reference)
import functools

import jax
import jax.numpy as jnp
from jax import lax
from jax.experimental import pallas as pl
from jax.experimental.pallas import tpu as pltpu
from jax.experimental.pallas import tpu_sc as plsc

D = 768
L = 16
NJ = D // L


def _bcast_total(v):
    lanes = lax.iota(jnp.int32, L)
    for k in (1, 2, 4, 8):
        perm = lanes ^ k
        v = v + v.at[perm].get(mode="promise_in_bounds", unique_indices=True)
    return v


def _rsqrt(x):
    i = plsc.bitcast(x, jnp.int32)
    i = jnp.int32(0x5F3759DF) - lax.shift_right_logical(i, 1)
    y = plsc.bitcast(i, jnp.float32)
    for _ in range(3):
        y = y * (1.5 - 0.5 * x * y * y)
    return y


def kernel(x, seg, tok_embed, pos_embed, seg_embed, ln_gamma, ln_beta):
    B, S = x.shape
    N = B * S

    info = plsc.get_sparse_core_info()
    NC, NS = info.num_cores, info.num_subcores
    NW = NC * NS
    n_per_w = N // NW
    K = 64
    n_chunks = n_per_w // K

    idx_tok = x.reshape(N).astype(jnp.int32)
    posseg = (pos_embed[:, None, :] + seg_embed[None, :, :]).reshape(S * 2, D)
    idx_ps = (jnp.arange(S, dtype=jnp.int32)[None, :] * 2
              + seg.astype(jnp.int32)).reshape(N)

    mesh = plsc.VectorSubcoreMesh(core_axis_name="c", subcore_axis_name="s")

    @functools.partial(
        pl.kernel,
        mesh=mesh,
        compiler_params=pltpu.CompilerParams(needs_layout_passes=False),
        out_type=jax.ShapeDtypeStruct((N, D), jnp.float32),
        scratch_types=[
            pltpu.VMEM((n_per_w,), jnp.int32),
            pltpu.VMEM((n_per_w,), jnp.int32),
            pltpu.VMEM((K, D), jnp.float32),
            pltpu.VMEM((K, D), jnp.float32),
            pltpu.VMEM((D,), jnp.float32),
            pltpu.VMEM((D,), jnp.float32),
            pltpu.SemaphoreType.DMA,
        ],
    )
    def sc_embed(tok_hbm, ps_hbm, idxt_hbm, idxp_hbm, g_hbm, b_hbm, out_hbm,
                 idxt_v, idxp_v, buf_a, buf_b, g_v, b_v, sem):
        wid = lax.axis_index("s") * NC + lax.axis_index("c")
        base = wid * n_per_w
        pltpu.sync_copy(idxt_hbm.at[pl.ds(base, n_per_w)], idxt_v)
        pltpu.sync_copy(idxp_hbm.at[pl.ds(base, n_per_w)], idxp_v)
        pltpu.sync_copy(g_hbm, g_v)
        pltpu.sync_copy(b_hbm, b_v)

        def chunk_body(c, carry):
            off = c * K
            ca = pltpu.async_copy(tok_hbm.at[idxt_v.at[pl.ds(off, K)]], buf_a, sem)
            cb = pltpu.async_copy(ps_hbm.at[idxp_v.at[pl.ds(off, K)]], buf_b, sem)
            ca.wait()
            cb.wait()

            def token_body(t, carry_t):
                def p1(j, acc):
                    s1, s2 = acc
                    v = buf_a[t, pl.ds(j * L, L)] + buf_b[t, pl.ds(j * L, L)]
                    buf_a[t, pl.ds(j * L, L)] = v
                    return s1 + v, s2 + v * v

                z = jnp.zeros((L,), jnp.float32)
                s1, s2 = lax.fori_loop(0, NJ, p1, (z, z))
                tot1 = _bcast_total(s1)
                tot2 = _bcast_total(s2)
                mean = tot1 * (1.0 / D)
                var = tot2 * (1.0 / D) - mean * mean
                r = _rsqrt(var + 1e-5)

                def p2(j, _):
                    v = buf_a[t, pl.ds(j * L, L)]
                    o = (v - mean) * r
                    o = o * g_v[pl.ds(j * L, L)] + b_v[pl.ds(j * L, L)]
                    buf_a[t, pl.ds(j * L, L)] = o
                    return 0

                lax.fori_loop(0, NJ, p2, 0)
                return 0

            lax.fori_loop(0, K, token_body, 0)
            pltpu.sync_copy(buf_a, out_hbm.at[pl.ds(base + off, K)])
            return 0

        lax.fori_loop(0, n_chunks, chunk_body, 0)

    out = sc_embed(tok_embed, posseg, idx_tok, idx_ps, ln_gamma, ln_beta)
    return out.reshape(B, S, D)


# speedup vs baseline: 1.8384x; 1.8384x over previous
import functools

import jax
import jax.numpy as jnp
from jax import lax
from jax.experimental import pallas as pl
from jax.experimental.pallas import tpu as pltpu
from jax.experimental.pallas import tpu_sc as plsc

D = 768
L = 16
NJ = D // L


def _bcast_total(v):
    lanes = lax.iota(jnp.int32, L)
    for k in (1, 2, 4, 8):
        perm = lanes ^ k
        v = v + v.at[perm].get(mode="promise_in_bounds", unique_indices=True)
    return v


def _rsqrt(x):
    i = plsc.bitcast(x, jnp.int32)
    i = jnp.int32(0x5F3759DF) - lax.shift_right_logical(i, 1)
    y = plsc.bitcast(i, jnp.float32)
    for _ in range(3):
        y = y * (1.5 - 0.5 * x * y * y)
    return y


def kernel(x, seg, tok_embed, pos_embed, seg_embed, ln_gamma, ln_beta):
    B, S = x.shape
    N = B * S

    info = plsc.get_sparse_core_info()
    NC, NS = info.num_cores, info.num_subcores
    NW = NC * NS
    n_per_w = N // NW
    K = 32
    n_chunks = n_per_w // K
    NBUF = 2

    idx_tok = x.reshape(N).astype(jnp.int32)
    posseg = (pos_embed[:, None, :] + seg_embed[None, :, :]).reshape(S * 2, D)
    idx_ps = (jnp.arange(S, dtype=jnp.int32)[None, :] * 2
              + seg.astype(jnp.int32)).reshape(N)

    mesh = plsc.VectorSubcoreMesh(core_axis_name="c", subcore_axis_name="s")

    @functools.partial(
        pl.kernel,
        mesh=mesh,
        compiler_params=pltpu.CompilerParams(needs_layout_passes=False),
        out_type=jax.ShapeDtypeStruct((N, D), jnp.float32),
        scratch_types=[
            pltpu.VMEM((n_per_w,), jnp.int32),
            pltpu.VMEM((n_per_w,), jnp.int32),
            pltpu.VMEM((NBUF, K, D), jnp.float32),
            pltpu.VMEM((NBUF, K, D), jnp.float32),
            pltpu.VMEM((D,), jnp.float32),
            pltpu.VMEM((D,), jnp.float32),
            pltpu.SemaphoreType.DMA((NBUF,)),
            pltpu.SemaphoreType.DMA((NBUF,)),
        ],
    )
    def sc_embed(tok_hbm, ps_hbm, idxt_hbm, idxp_hbm, g_hbm, b_hbm, out_hbm,
                 idxt_v, idxp_v, buf_tok, buf_ps, g_v, b_v, sem_g, sem_o):
        wid = lax.axis_index("s") * NC + lax.axis_index("c")
        base = wid * n_per_w
        pltpu.sync_copy(idxt_hbm.at[pl.ds(base, n_per_w)], idxt_v)
        pltpu.sync_copy(idxp_hbm.at[pl.ds(base, n_per_w)], idxp_v)
        pltpu.sync_copy(g_hbm, g_v)
        pltpu.sync_copy(b_hbm, b_v)

        def issue_gathers(c, slot):
            off = c * K
            pltpu.async_copy(tok_hbm.at[idxt_v.at[pl.ds(off, K)]],
                             buf_tok.at[slot], sem_g.at[slot])
            pltpu.async_copy(ps_hbm.at[idxp_v.at[pl.ds(off, K)]],
                             buf_ps.at[slot], sem_g.at[slot])

        def wait_gathers(c, slot):
            off = c * K
            pltpu.make_async_copy(tok_hbm.at[idxt_v.at[pl.ds(off, K)]],
                                  buf_tok.at[slot], sem_g.at[slot]).wait()
            pltpu.make_async_copy(ps_hbm.at[idxp_v.at[pl.ds(off, K)]],
                                  buf_ps.at[slot], sem_g.at[slot]).wait()

        def compute_chunk(slot):
            bt = buf_tok.at[slot]
            bp = buf_ps.at[slot]

            def token_body(t, _):
                z = jnp.zeros((L,), jnp.float32)
                s1 = [z, z, z, z]
                s2 = [z, z, z, z]
                for j in range(NJ):
                    v = bt[t, pl.ds(j * L, L)] + bp[t, pl.ds(j * L, L)]
                    bt[t, pl.ds(j * L, L)] = v
                    s1[j % 4] = s1[j % 4] + v
                    s2[j % 4] = s2[j % 4] + v * v
                tot1 = _bcast_total((s1[0] + s1[1]) + (s1[2] + s1[3]))
                tot2 = _bcast_total((s2[0] + s2[1]) + (s2[2] + s2[3]))
                mean = tot1 * (1.0 / D)
                var = tot2 * (1.0 / D) - mean * mean
                r = _rsqrt(var + 1e-5)
                for j in range(NJ):
                    v = bt[t, pl.ds(j * L, L)]
                    o = (v - mean) * r
                    o = o * g_v[pl.ds(j * L, L)] + b_v[pl.ds(j * L, L)]
                    bt[t, pl.ds(j * L, L)] = o
                return 0

            lax.fori_loop(0, K, token_body, 0, unroll=False)

        def issue_out(c, slot):
            pltpu.async_copy(buf_tok.at[slot],
                             out_hbm.at[pl.ds(base + c * K, K)], sem_o.at[slot])

        def wait_out(c, slot):
            pltpu.make_async_copy(buf_tok.at[slot],
                                  out_hbm.at[pl.ds(base + c * K, K)],
                                  sem_o.at[slot]).wait()

        issue_gathers(0, 0)

        def group_body(g, _):
            for b in range(NBUF):
                c = g * NBUF + b
                nxt = (b + 1) % NBUF
                @pl.when(c >= 1)
                def _():
                    wait_out(c - 1, nxt)

                @pl.when(c + 1 < n_chunks)
                def _():
                    issue_gathers(c + 1, nxt)

                wait_gathers(c, b)
                compute_chunk(b)
                issue_out(c, b)
            return 0

        lax.fori_loop(0, n_chunks // NBUF, group_body, 0, unroll=False)
        wait_out(n_chunks - 1, (n_chunks - 1) % NBUF)

    out = sc_embed(tok_embed, posseg, idx_tok, idx_ps, ln_gamma, ln_beta)
    return out.reshape(B, S, D)


# speedup vs baseline: 3.2079x; 1.7449x over previous
import functools

import jax
import jax.numpy as jnp
from jax import lax
from jax.experimental import pallas as pl
from jax.experimental.pallas import tpu as pltpu
from jax.experimental.pallas import tpu_sc as plsc

D = 768
L = 16
NJ = D // L


def _bcast_total(v):
    lanes = lax.iota(jnp.int32, L)
    for k in (1, 2, 4, 8):
        perm = lanes ^ k
        v = v + v.at[perm].get(mode="promise_in_bounds", unique_indices=True)
    return v


def _rsqrt(x):
    i = plsc.bitcast(x, jnp.int32)
    i = jnp.int32(0x5F3759DF) - lax.shift_right_logical(i, 1)
    y = plsc.bitcast(i, jnp.float32)
    for _ in range(2):
        y = y * (1.5 - 0.5 * x * y * y)
    return y


def kernel(x, seg, tok_embed, pos_embed, seg_embed, ln_gamma, ln_beta):
    B, S = x.shape
    N = B * S

    info = plsc.get_sparse_core_info()
    NC, NS = info.num_cores, info.num_subcores
    NW = NC * NS
    n_per_w = N // NW
    K = 16
    n_chunks = n_per_w // K
    NBUF = 2

    idx_tok = x.reshape(N).astype(jnp.int32)
    posseg = (pos_embed[:, None, :] + seg_embed[None, :, :]).reshape(S * 2, D)
    idx_ps = (jnp.arange(S, dtype=jnp.int32)[None, :] * 2
              + seg.astype(jnp.int32)).reshape(N)

    mesh = plsc.VectorSubcoreMesh(core_axis_name="c", subcore_axis_name="s")

    @functools.partial(
        pl.kernel,
        mesh=mesh,
        compiler_params=pltpu.CompilerParams(needs_layout_passes=False),
        out_type=jax.ShapeDtypeStruct((N, D), jnp.float32),
        scratch_types=[
            pltpu.VMEM((n_per_w,), jnp.int32),
            pltpu.VMEM((n_per_w,), jnp.int32),
            pltpu.VMEM((NBUF, K, D), jnp.float32),
            pltpu.VMEM((NBUF, K, D), jnp.float32),
            pltpu.VMEM((K, D), jnp.float32),
            pltpu.VMEM((K, D), jnp.float32),
            pltpu.VMEM((NBUF, K, D), jnp.float32),
            pltpu.VMEM((D,), jnp.float32),
            pltpu.VMEM((D,), jnp.float32),
            pltpu.SemaphoreType.DMA((NBUF,)),
            pltpu.SemaphoreType.DMA((NBUF,)),
        ],
    )
    def sc_embed(tok_hbm, ps_hbm, idxt_hbm, idxp_hbm, g_hbm, b_hbm, out_hbm,
                 idxt_v, idxp_v, buf_tok, buf_ps, buf_v, buf_n, buf_o,
                 g_v, b_v, sem_g, sem_o):
        wid = lax.axis_index("s") * NC + lax.axis_index("c")
        base = wid * n_per_w
        pltpu.sync_copy(idxt_hbm.at[pl.ds(base, n_per_w)], idxt_v)
        pltpu.sync_copy(idxp_hbm.at[pl.ds(base, n_per_w)], idxp_v)
        pltpu.sync_copy(g_hbm, g_v)
        pltpu.sync_copy(b_hbm, b_v)

        def issue_gathers(c, slot):
            off = c * K
            pltpu.async_copy(tok_hbm.at[idxt_v.at[pl.ds(off, K)]],
                             buf_tok.at[slot], sem_g.at[slot])
            pltpu.async_copy(ps_hbm.at[idxp_v.at[pl.ds(off, K)]],
                             buf_ps.at[slot], sem_g.at[slot])

        def wait_gathers(c, slot):
            off = c * K
            pltpu.make_async_copy(tok_hbm.at[idxt_v.at[pl.ds(off, K)]],
                                  buf_tok.at[slot], sem_g.at[slot]).wait()
            pltpu.make_async_copy(ps_hbm.at[idxp_v.at[pl.ds(off, K)]],
                                  buf_ps.at[slot], sem_g.at[slot]).wait()

        def compute_chunk(slot):
            bt = buf_tok.at[slot]
            bp = buf_ps.at[slot]
            bo = buf_o.at[slot]

            def token_body(t, _):
                z = jnp.zeros((L,), jnp.float32)
                s1 = [z, z, z, z]
                s2 = [z, z, z, z]
                for j in range(NJ):
                    v = bt[t, pl.ds(j * L, L)] + bp[t, pl.ds(j * L, L)]
                    buf_v[t, pl.ds(j * L, L)] = v
                    s1[j % 4] = s1[j % 4] + v
                    s2[j % 4] = s2[j % 4] + v * v
                tot1 = _bcast_total((s1[0] + s1[1]) + (s1[2] + s1[3]))
                tot2 = _bcast_total((s2[0] + s2[1]) + (s2[2] + s2[3]))
                mean = tot1 * (1.0 / D)
                var = tot2 * (1.0 / D) - mean * mean
                r = _rsqrt(var + 1e-5)
                for j in range(NJ):
                    buf_n[t, pl.ds(j * L, L)] = (buf_v[t, pl.ds(j * L, L)]
                                                 - mean) * r
                return 0

            lax.fori_loop(0, K, token_body, 0, unroll=False)

            def gb_body(j, _):
                gj = g_v[pl.ds(j * L, L)]
                bj = b_v[pl.ds(j * L, L)]
                for t in range(K):
                    bo[t, pl.ds(j * L, L)] = buf_n[t, pl.ds(j * L, L)] * gj + bj
                return 0

            lax.fori_loop(0, NJ, gb_body, 0, unroll=False)

        def issue_out(c, slot):
            pltpu.async_copy(buf_o.at[slot],
                             out_hbm.at[pl.ds(base + c * K, K)], sem_o.at[slot])

        def wait_out(c, slot):
            pltpu.make_async_copy(buf_o.at[slot],
                                  out_hbm.at[pl.ds(base + c * K, K)],
                                  sem_o.at[slot]).wait()

        issue_gathers(0, 0)

        def group_body(g, _):
            for b in range(NBUF):
                c = g * NBUF + b
                nxt = (b + 1) % NBUF
                @pl.when(c >= 1)
                def _():
                    wait_out(c - 1, nxt)

                @pl.when(c + 1 < n_chunks)
                def _():
                    issue_gathers(c + 1, nxt)

                wait_gathers(c, b)
                compute_chunk(b)
                issue_out(c, b)
            return 0

        lax.fori_loop(0, n_chunks // NBUF, group_body, 0, unroll=False)
        wait_out(n_chunks - 1, (n_chunks - 1) % NBUF)

    out = sc_embed(tok_embed, posseg, idx_tok, idx_ps, ln_gamma, ln_beta)
    return out.reshape(B, S, D)
